# baseline (speedup 1.0000x reference)
import functools

import jax
import jax.numpy as jnp
from jax import lax
from jax.experimental import pallas as pl
from jax.experimental.pallas import tpu as pltpu
from jax.experimental.pallas import tpu_sc as plsc

_NC = 2
_NS = 16
_NW = _NC * _NS


def _lstm_step(p_t, h, c, u):
    Hs = h.shape[1]
    g = p_t + lax.dot_general(h, u, (((1,), (1,)), ((), ())),
                              preferred_element_type=jnp.float32)
    i_ = jax.nn.sigmoid(g[:, 0:Hs])
    f_ = jax.nn.sigmoid(g[:, Hs:2 * Hs])
    g_ = jnp.tanh(g[:, 2 * Hs:3 * Hs])
    o_ = jax.nn.sigmoid(g[:, 3 * Hs:4 * Hs])
    c = f_ * c + i_ * g_
    h = o_ * jnp.tanh(c)
    return h, c


def _lstm_body(pt_ref, w0f, u0f, w0r, u0r, w1f, u1f, w1r, u1r,
               b0f, b0r, b1f, b1r,
               out_ref, pf_s, pr_s, hf_s, hr_s):
    P = pt_ref.shape[0]
    H = u0f.shape[1]
    x = pt_ref[...]
    for (wf, uf, wr, ur, bf, br) in (
        (w0f, u0f, w0r, u0r, b0f, b0r),
        (w1f, u1f, w1r, u1r, b1f, b1r),
    ):
        pf = lax.dot_general(x, wf[...], (((1,), (1,)), ((), ())),
                             preferred_element_type=jnp.float32) + bf[...]
        pr = lax.dot_general(x, wr[...], (((1,), (1,)), ((), ())),
                             preferred_element_type=jnp.float32) + br[...]
        pf_s[0:P, :] = pf
        pr_s[0:P, :] = pr
        uf_v = uf[...]
        ur_v = ur[...]

        def step(t, carry):
            hf, cf, hr, cr = carry
            hf, cf = _lstm_step(pf_s[pl.ds(t, 1), :], hf, cf, uf_v)
            hf_s[pl.ds(t, 1), :] = hf[0:1, :]
            tr = P - 1 - t
            hr, cr = _lstm_step(pr_s[pl.ds(tr, 1), :], hr, cr, ur_v)
            hr_s[pl.ds(tr, 1), :] = hr[0:1, :]
            return hf, cf, hr, cr

        z = jnp.zeros((8, H), jnp.float32)
        lax.fori_loop(0, P, step, (z, z, z, z))
        x = jnp.concatenate([hf_s[0:P, :], hr_s[0:P, :]], axis=1)

    out_ref[...] = x


def _mlp_body(x_ref, w_mlp1, b_mlp1, w_mlp2, b_mlp2, out_ref):
    y = lax.dot_general(x_ref[...], w_mlp1[...], (((1,), (1,)), ((), ())),
                        preferred_element_type=jnp.float32) + b_mlp1[...]
    y = jnp.maximum(y, 0.0)
    y = lax.dot_general(y, w_mlp2[...], (((1,), (1,)), ((), ())),
                        preferred_element_type=jnp.float32) + b_mlp2[...]
    out_ref[...] = y


def _prompt_encode(prompt_table,
                   Wih_l0f, Whh_l0f, bih_l0f, bhh_l0f,
                   Wih_l0r, Whh_l0r, bih_l0r, bhh_l0r,
                   Wih_l1f, Whh_l1f, bih_l1f, bhh_l1f,
                   Wih_l1r, Whh_l1r, bih_l1r, bhh_l1r,
                   W1, b1, W2, b2):
    P, D = prompt_table.shape
    H4 = Wih_l0f.shape[0]
    b0f = (bih_l0f + bhh_l0f).reshape(1, H4)
    b0r = (bih_l0r + bhh_l0r).reshape(1, H4)
    b1f = (bih_l1f + bhh_l1f).reshape(1, H4)
    b1r = (bih_l1r + bhh_l1r).reshape(1, H4)
    x = pl.pallas_call(
        _lstm_body,
        out_shape=jax.ShapeDtypeStruct((P, 2 * (H4 // 4)), jnp.float32),
        scratch_shapes=[
            pltpu.VMEM((P, H4), jnp.float32),
            pltpu.VMEM((P, H4), jnp.float32),
            pltpu.VMEM((P, H4 // 4), jnp.float32),
            pltpu.VMEM((P, H4 // 4), jnp.float32),
        ],
        name="prompt_lstm",
    )(prompt_table,
      Wih_l0f, Whh_l0f, Wih_l0r, Whh_l0r,
      Wih_l1f, Whh_l1f, Wih_l1r, Whh_l1r,
      b0f, b0r, b1f, b1r)
    return pl.pallas_call(
        _mlp_body,
        out_shape=jax.ShapeDtypeStruct((P, D), jnp.float32),
        name="prompt_mlp",
    )(x, W1, b1.reshape(1, D), W2, b2.reshape(1, D))


_CHUNK = 32


def _gather_body(rows_per_w, ids_hbm, emb_hbm, out_hbm,
                 idx_v, rows_a, rows_b, isem_a, isem_b, osem_a, osem_b):
    wid = lax.axis_index("s") * _NC + lax.axis_index("c")
    base = wid * rows_per_w
    pltpu.sync_copy(ids_hbm.at[pl.ds(base, rows_per_w)], idx_v)
    nk = rows_per_w // _CHUNK
    bufs = (rows_a, rows_b)
    isems = (isem_a, isem_b)
    osems = (osem_a, osem_b)
    gathers = [None, None]
    stores = [None, None]
    gathers[0] = pltpu.async_copy(
        emb_hbm.at[idx_v.at[pl.ds(0, _CHUNK)]], rows_a, isem_a)
    for k in range(nk):
        j = k % 2
        nj = (k + 1) % 2
        gathers[j].wait()
        stores[j] = pltpu.async_copy(
            bufs[j], out_hbm.at[pl.ds(base + k * _CHUNK, _CHUNK)], osems[j])
        if k + 1 < nk:
            if stores[nj] is not None:
                stores[nj].wait()
            gathers[nj] = pltpu.async_copy(
                emb_hbm.at[idx_v.at[pl.ds((k + 1) * _CHUNK, _CHUNK)]],
                bufs[nj], isems[nj])
    stores[(nk - 1) % 2].wait()
    if stores[nk % 2] is not None:
        stores[nk % 2].wait()


def _sc_gather(flat_ids, emb_table):
    N = flat_ids.shape[0]
    D = emb_table.shape[1]
    rows_per_w = N // _NW
    mesh = plsc.VectorSubcoreMesh(core_axis_name="c", subcore_axis_name="s")
    return pl.kernel(
        functools.partial(_gather_body, rows_per_w),
        out_type=jax.ShapeDtypeStruct((N, D), jnp.float32),
        mesh=mesh,
        scratch_types=[
            pltpu.VMEM((rows_per_w,), jnp.int32),
            pltpu.VMEM((_CHUNK, D), jnp.float32),
            pltpu.VMEM((_CHUNK, D), jnp.float32),
            pltpu.SemaphoreType.DMA,
            pltpu.SemaphoreType.DMA,
            pltpu.SemaphoreType.DMA,
            pltpu.SemaphoreType.DMA,
        ],
        name="sc_emb_gather",
    )(flat_ids, emb_table)


def _overwrite_body(pi_ref, x_ref, _, out_ref):
    del pi_ref
    out_ref[...] = x_ref[...]


def _scatter_prompts(gathered_flat, x, flat_pi):
    N, D = gathered_flat.shape
    P = x.shape[0]
    BP = flat_pi.shape[0]
    grid_spec = pltpu.PrefetchScalarGridSpec(
        num_scalar_prefetch=1,
        grid=(BP,),
        in_specs=[
            pl.BlockSpec((1, 1, D), lambda i, pi: (lax.rem(i, P), 0, 0)),
            pl.BlockSpec((1, 1, D), lambda i, pi: (pi[i], 0, 0)),
        ],
        out_specs=pl.BlockSpec((1, 1, D), lambda i, pi: (pi[i], 0, 0)),
    )
    out = pl.pallas_call(
        _overwrite_body,
        grid_spec=grid_spec,
        out_shape=jax.ShapeDtypeStruct((N, 1, D), jnp.float32),
        input_output_aliases={2: 0},
        name="prompt_overwrite",
    )(flat_pi, x.reshape(P, 1, D), gathered_flat.reshape(N, 1, D))
    return out.reshape(N, D)


def kernel(input_ids, prompt_indexes, emb_table, prompt_table,
           Wih_l0f, Whh_l0f, bih_l0f, bhh_l0f,
           Wih_l0r, Whh_l0r, bih_l0r, bhh_l0r,
           Wih_l1f, Whh_l1f, bih_l1f, bhh_l1f,
           Wih_l1r, Whh_l1r, bih_l1r, bhh_l1r,
           W1, b1, W2, b2):
    B, S = input_ids.shape
    P, D = prompt_table.shape
    x = _prompt_encode(prompt_table,
                       Wih_l0f, Whh_l0f, bih_l0f, bhh_l0f,
                       Wih_l0r, Whh_l0r, bih_l0r, bhh_l0r,
                       Wih_l1f, Whh_l1f, bih_l1f, bhh_l1f,
                       Wih_l1r, Whh_l1r, bih_l1r, bhh_l1r,
                       W1, b1, W2, b2)
    flat_ids = input_ids.astype(jnp.int32).reshape(B * S)
    gathered = _sc_gather(flat_ids, emb_table)
    flat_pi = (jnp.arange(B, dtype=jnp.int32)[:, None] * S
               + prompt_indexes.astype(jnp.int32)).reshape(B * P)
    out = _scatter_prompts(gathered, x, flat_pi)
    return out.reshape(B, S, D)


# speedup vs baseline: 2.0657x; 2.0657x over previous
import functools

import jax
import jax.numpy as jnp
from jax import lax
from jax.experimental import pallas as pl
from jax.experimental.pallas import tpu as pltpu
from jax.experimental.pallas import tpu_sc as plsc

_NC = 2
_NS = 16
_NW = _NC * _NS


_CHUNK = 32


def _gather_body(rows_per_w, ids_hbm, emb_hbm, out_hbm,
                 idx_v, rows_a, rows_b, isem_a, isem_b, osem_a, osem_b):
    wid = lax.axis_index("s") * _NC + lax.axis_index("c")
    base = wid * rows_per_w
    pltpu.sync_copy(ids_hbm.at[pl.ds(base, rows_per_w)], idx_v)
    nk = rows_per_w // _CHUNK
    bufs = (rows_a, rows_b)
    isems = (isem_a, isem_b)
    osems = (osem_a, osem_b)
    gathers = [None, None]
    stores = [None, None]
    gathers[0] = pltpu.async_copy(
        emb_hbm.at[idx_v.at[pl.ds(0, _CHUNK)]], rows_a, isem_a)
    for k in range(nk):
        j = k % 2
        nj = (k + 1) % 2
        gathers[j].wait()
        stores[j] = pltpu.async_copy(
            bufs[j], out_hbm.at[pl.ds(base + k * _CHUNK, _CHUNK)], osems[j])
        if k + 1 < nk:
            if stores[nj] is not None:
                stores[nj].wait()
            gathers[nj] = pltpu.async_copy(
                emb_hbm.at[idx_v.at[pl.ds((k + 1) * _CHUNK, _CHUNK)]],
                bufs[nj], isems[nj])
    stores[(nk - 1) % 2].wait()
    if stores[nk % 2] is not None:
        stores[nk % 2].wait()


def _sc_gather(flat_ids, emb_table):
    N = flat_ids.shape[0]
    D = emb_table.shape[1]
    rows_per_w = N // _NW
    mesh = plsc.VectorSubcoreMesh(core_axis_name="c", subcore_axis_name="s")
    return pl.kernel(
        functools.partial(_gather_body, rows_per_w),
        out_type=jax.ShapeDtypeStruct((N, D), jnp.float32),
        mesh=mesh,
        scratch_types=[
            pltpu.VMEM((rows_per_w,), jnp.int32),
            pltpu.VMEM((_CHUNK, D), jnp.float32),
            pltpu.VMEM((_CHUNK, D), jnp.float32),
            pltpu.SemaphoreType.DMA,
            pltpu.SemaphoreType.DMA,
            pltpu.SemaphoreType.DMA,
            pltpu.SemaphoreType.DMA,
        ],
        name="sc_emb_gather",
    )(flat_ids, emb_table)


def _lstm_step(p_t, h, c, u):
    Hs = h.shape[1]
    g = p_t + lax.dot_general(h, u, (((1,), (1,)), ((), ())),
                              preferred_element_type=jnp.float32)
    i_ = jax.nn.sigmoid(g[:, 0:Hs])
    f_ = jax.nn.sigmoid(g[:, Hs:2 * Hs])
    g_ = jnp.tanh(g[:, 2 * Hs:3 * Hs])
    o_ = jax.nn.sigmoid(g[:, 3 * Hs:4 * Hs])
    c = f_ * c + i_ * g_
    h = o_ * jnp.tanh(c)
    return h, c


def _lstm_body(pt_ref, w0f, u0f, w0r, u0r, w1f, u1f, w1r, u1r,
               b0f, b0r, b1f, b1r, out_ref,
               w0f_s, u0f_s, w0r_s, u0r_s, w1f_s, u1f_s, w1r_s, u1r_s,
               pf_s, pr_s, hf_s, hr_s,
               s0, s1, s2, s3, s4, s5, s6, s7):
    P = pt_ref.shape[0]
    H = u0f_s.shape[1]
    srcs = (w0f, w0r, u0f, u0r, w1f, w1r, u1f, u1r)
    dsts = (w0f_s, w0r_s, u0f_s, u0r_s, w1f_s, w1r_s, u1f_s, u1r_s)
    sems = (s0, s1, s2, s3, s4, s5, s6, s7)
    cps = []
    for src, dst, sem in zip(srcs, dsts, sems):
        c = pltpu.make_async_copy(src, dst, sem)
        c.start()
        cps.append(c)
    x = pt_ref[...]
    for li, (wf_s, wr_s, uf_s, ur_s, bf, br) in enumerate((
        (w0f_s, w0r_s, u0f_s, u0r_s, b0f, b0r),
        (w1f_s, w1r_s, u1f_s, u1r_s, b1f, b1r),
    )):
        o = li * 4
        cps[o + 0].wait()
        pf = lax.dot_general(x, wf_s[...], (((1,), (1,)), ((), ())),
                             preferred_element_type=jnp.float32) + bf[...]
        pf_s[0:P, :] = pf
        cps[o + 1].wait()
        pr = lax.dot_general(x, wr_s[...], (((1,), (1,)), ((), ())),
                             preferred_element_type=jnp.float32) + br[...]
        pr_s[0:P, :] = pr
        cps[o + 2].wait()
        cps[o + 3].wait()
        uf_v = uf_s[...]
        ur_v = ur_s[...]

        def step(t, carry):
            hf, cf, hr, cr = carry
            hf, cf = _lstm_step(pf_s[pl.ds(t, 1), :], hf, cf, uf_v)
            hf_s[pl.ds(t, 1), :] = hf[0:1, :]
            tr = P - 1 - t
            hr, cr = _lstm_step(pr_s[pl.ds(tr, 1), :], hr, cr, ur_v)
            hr_s[pl.ds(tr, 1), :] = hr[0:1, :]
            return hf, cf, hr, cr

        z = jnp.zeros((8, H), jnp.float32)
        lax.fori_loop(0, P, step, (z, z, z, z))
        x = jnp.concatenate([hf_s[0:P, :], hr_s[0:P, :]], axis=1)

    out_ref[...] = x


def _lstm_encode(prompt_table,
                 Wih_l0f, Whh_l0f, b0f, Wih_l0r, Whh_l0r, b0r,
                 Wih_l1f, Whh_l1f, b1f, Wih_l1r, Whh_l1r, b1r):
    P, D = prompt_table.shape
    H4 = Wih_l0f.shape[0]
    H = H4 // 4
    anyspec = pl.BlockSpec(memory_space=pl.ANY)
    vmem = pl.BlockSpec(memory_space=pltpu.VMEM)
    return pl.pallas_call(
        _lstm_body,
        out_shape=jax.ShapeDtypeStruct((P, 2 * H), jnp.float32),
        in_specs=[vmem,
                  anyspec, anyspec, anyspec, anyspec,
                  anyspec, anyspec, anyspec, anyspec,
                  vmem, vmem, vmem, vmem],
        out_specs=vmem,
        scratch_shapes=[
            pltpu.VMEM((H4, D), jnp.float32),
            pltpu.VMEM((H4, H), jnp.float32),
            pltpu.VMEM((H4, D), jnp.float32),
            pltpu.VMEM((H4, H), jnp.float32),
            pltpu.VMEM((H4, 2 * H), jnp.float32),
            pltpu.VMEM((H4, H), jnp.float32),
            pltpu.VMEM((H4, 2 * H), jnp.float32),
            pltpu.VMEM((H4, H), jnp.float32),
            pltpu.VMEM((P, H4), jnp.float32),
            pltpu.VMEM((P, H4), jnp.float32),
            pltpu.VMEM((P, H), jnp.float32),
            pltpu.VMEM((P, H), jnp.float32),
        ] + [pltpu.SemaphoreType.DMA] * 8,
        name="prompt_lstm",
    )(prompt_table,
      Wih_l0f, Whh_l0f, Wih_l0r, Whh_l0r,
      Wih_l1f, Whh_l1f, Wih_l1r, Whh_l1r,
      b0f, b0r, b1f, b1r)


def _mlp_scatter_body(n_rows, x_ref, w1, b1_, w2, b2_, pidx_ref, gath_any,
                      out_any, y_s, sem):
    del gath_any
    P = x_ref.shape[0]
    y = lax.dot_general(x_ref[...], w1[...], (((1,), (1,)), ((), ())),
                        preferred_element_type=jnp.float32) + b1_[...]
    y = jnp.maximum(y, 0.0)
    y = lax.dot_general(y, w2[...], (((1,), (1,)), ((), ())),
                        preferred_element_type=jnp.float32) + b2_[...]
    y_s[...] = y
    cps = []
    for j in range(n_rows):
        row = pidx_ref[j]
        c = pltpu.make_async_copy(
            y_s.at[pl.ds(j % P, 1)], out_any.at[pl.ds(row, 1)], sem)
        c.start()
        cps.append(c)
    for c in cps:
        c.wait()


def _mlp_scatter(x, W1, b1, W2, b2, flat_pi, gathered):
    P, D = x.shape
    BP = flat_pi.shape[0]
    N = gathered.shape[0]
    anyspec = pl.BlockSpec(memory_space=pl.ANY)
    vmem = pl.BlockSpec(memory_space=pltpu.VMEM)
    smem = pl.BlockSpec(memory_space=pltpu.SMEM)
    return pl.pallas_call(
        functools.partial(_mlp_scatter_body, BP),
        out_shape=jax.ShapeDtypeStruct((N, D), jnp.float32),
        in_specs=[vmem, vmem, vmem, vmem, vmem, smem, anyspec],
        out_specs=anyspec,
        scratch_shapes=[
            pltpu.VMEM((P, D), jnp.float32),
            pltpu.SemaphoreType.DMA,
        ],
        input_output_aliases={6: 0},
        name="prompt_mlp_scatter",
    )(x, W1, b1.reshape(1, D), W2, b2.reshape(1, D), flat_pi, gathered)


def kernel(input_ids, prompt_indexes, emb_table, prompt_table,
           Wih_l0f, Whh_l0f, bih_l0f, bhh_l0f,
           Wih_l0r, Whh_l0r, bih_l0r, bhh_l0r,
           Wih_l1f, Whh_l1f, bih_l1f, bhh_l1f,
           Wih_l1r, Whh_l1r, bih_l1r, bhh_l1r,
           W1, b1, W2, b2):
    B, S = input_ids.shape
    P, D = prompt_table.shape
    H4 = Wih_l0f.shape[0]
    b0f = (bih_l0f + bhh_l0f).reshape(1, H4)
    b0r = (bih_l0r + bhh_l0r).reshape(1, H4)
    b1f = (bih_l1f + bhh_l1f).reshape(1, H4)
    b1r = (bih_l1r + bhh_l1r).reshape(1, H4)
    x = _lstm_encode(prompt_table,
                     Wih_l0f, Whh_l0f, b0f, Wih_l0r, Whh_l0r, b0r,
                     Wih_l1f, Whh_l1f, b1f, Wih_l1r, Whh_l1r, b1r)
    flat_ids = input_ids.astype(jnp.int32).reshape(B * S)
    gathered = _sc_gather(flat_ids, emb_table)
    flat_pi = (jnp.arange(B, dtype=jnp.int32)[:, None] * S
               + prompt_indexes.astype(jnp.int32)).reshape(B * P)
    out = _mlp_scatter(x, W1, b1, W2, b2, flat_pi, gathered)
    return out.reshape(B, S, D)
